# baseline (speedup 1.0000x reference)
import jax
import jax.numpy as jnp
from jax import lax
from jax.experimental import pallas as pl
from jax.experimental.pallas import tpu as pltpu
from jax.experimental.pallas import tpu_sc as plsc

_F = 26
_V = 100000
_E = 32
_L = 16
_NC = 2
_NS = 16
_NW = _NC * _NS
_B = 16384
_RPW = _B // _NW
_FP = 32

_TCV = 1024
_TCC = 98
_VP = _TCC * _TCV
_SROWS = _F * _TCC * 8

_R = 64
_NCH = _RPW // _R
_CIDX = _R * _FP
_SG = 128
_NSG = _CIDX // _SG
_IDXW = _RPW * _FP


def _scores_body(tt_ref, w_ref, s_ref):
    f_i = pl.program_id(0)
    t2 = tt_ref[0]
    sv = t2[0] * w_ref[f_i, 0]
    for e in range(1, _E):
        sv = sv + t2[e] * w_ref[f_i, e]
    s_ref[...] = sv.reshape(8, 128)


def _combine_body(s_ref, xcat_ref, xcont_ref, wc_ref, out_ref,
                  idx_v, g0, g1, xcont_v, out_v, wc_v, sem0, sem1):
    wid = lax.axis_index("s") * _NC + lax.axis_index("c")
    base = wid * _RPW

    pltpu.sync_copy(xcat_ref.at[pl.ds(base * _FP, _IDXW)], idx_v)
    pltpu.sync_copy(xcont_ref.at[pl.ds(base * _L, _RPW * _L)], xcont_v)
    pltpu.sync_copy(wc_ref, wc_v)

    lanes = lax.iota(jnp.int32, _L)

    def build_idx(j, carry):
        sl = pl.ds(j * _L, _L)
        f = (j * _L + lanes) & (_FP - 1)
        t = idx_v[sl] + f * _VP
        idx_v[sl] = jnp.where(f < _F, t, 0)
        return carry

    lax.fori_loop(0, _IDXW // _L, build_idx, 0)

    bufs = (g0, g1)
    sems = (sem0, sem1)

    def fire(ci, k):
        for g in range(_NSG):
            isl = pl.ds(ci * _CIDX + g * _SG, _SG)
            pltpu.async_copy(s_ref.at[idx_v.at[isl]],
                             bufs[k].at[pl.ds(g * _SG, _SG)], sems[k])

    def drain(ci, k):
        for g in range(_NSG):
            isl = pl.ds(ci * _CIDX + g * _SG, _SG)
            pltpu.make_async_copy(s_ref.at[idx_v.at[isl]],
                                  bufs[k].at[pl.ds(g * _SG, _SG)], sems[k]).wait()

    def lane_sum(v):
        for s in (8, 4, 2, 1):
            idx = (lanes ^ s)[:, None]
            dn = lax.GatherDimensionNumbers(
                offset_dims=(), collapsed_slice_dims=(0,), start_index_map=(0,))
            v = v + lax.gather(v, idx, dn, (1,),
                               mode=lax.GatherScatterMode.PROMISE_IN_BOUNDS)
        return v

    wc = wc_v[...]
    mask26 = lanes < (_F - _L)
    zero = jnp.zeros((_L,), jnp.float32)

    def compute(ci, k):
        rbuf = bufs[k]

        def group_body(g2, carry):
            def row_body(r2, tv):
                r = g2 * _L + r2
                v0 = rbuf[pl.ds(r * _FP, _L)]
                v1 = rbuf[pl.ds(r * _FP + _L, _L)]
                xc = xcont_v[pl.ds((ci * _R + r) * _L, _L)]
                acc = v0 + jnp.where(mask26, v1, zero) + xc * wc
                tot = lane_sum(acc)
                return jnp.where(lanes == r2, tot, tv)

            tv = lax.fori_loop(0, _L, row_body, zero)
            out_v[pl.ds(ci * _R + g2 * _L, _L)] = 1.0 / (1.0 + jnp.exp(-tv))
            return carry

        lax.fori_loop(0, _R // _L, group_body, 0)

    fire(0, 0)

    def outer(c2, carry):
        for k in range(2):
            i = c2 * 2 + k
            drain(i, k)

            @pl.when(i + 1 < _NCH)
            def _():
                fire(i + 1, k ^ 1)

            compute(i, k)
        return carry

    lax.fori_loop(0, _NCH // 2, outer, 0)

    pltpu.sync_copy(out_v, out_ref.at[pl.ds(base, _RPW)])


@jax.jit
def _run(tt, w2d, xcat32_flat, xcont_flat, wc_pad):
    s = pl.pallas_call(
        _scores_body,
        grid=(_F, _TCC),
        in_specs=[
            pl.BlockSpec((1, _E, _TCV), lambda f, c: (f, 0, c)),
            pl.BlockSpec(memory_space=pltpu.MemorySpace.SMEM),
        ],
        out_specs=pl.BlockSpec((8, 128), lambda f, c: (f * _TCC + c, 0)),
        out_shape=jax.ShapeDtypeStruct((_SROWS, 128), jnp.float32),
    )(tt, w2d)

    combine = pl.kernel(
        _combine_body,
        out_type=jax.ShapeDtypeStruct((_B,), jnp.float32),
        mesh=plsc.VectorSubcoreMesh(core_axis_name="c", subcore_axis_name="s",
                                    num_cores=_NC, num_subcores=_NS),
        compiler_params=pltpu.CompilerParams(use_tc_tiling_on_sc=False),
        scratch_types=[
            pltpu.VMEM((_IDXW,), jnp.int32),
            pltpu.VMEM((_CIDX,), jnp.float32),
            pltpu.VMEM((_CIDX,), jnp.float32),
            pltpu.VMEM((_RPW * _L,), jnp.float32),
            pltpu.VMEM((_RPW,), jnp.float32),
            pltpu.VMEM((_L,), jnp.float32),
            pltpu.SemaphoreType.DMA,
            pltpu.SemaphoreType.DMA,
        ],
    )
    return combine(s.reshape(-1), xcat32_flat, xcont_flat, wc_pad)


def kernel(x_cat, x_cont, tables, W, b):
    bsz = x_cat.shape[0]
    tt = jnp.transpose(tables, (0, 2, 1))
    w2d = W[: _F * _E, 0].reshape(_F, _E)
    xcat32 = jnp.concatenate(
        [x_cat, jnp.zeros((bsz, _FP - _F), jnp.int32)], axis=1)
    xcont_pad = jnp.concatenate(
        [x_cont, jnp.ones((bsz, 1), jnp.float32), jnp.zeros((bsz, 2), jnp.float32)],
        axis=1)
    wc_pad = jnp.concatenate([W[_F * _E:, 0], b, jnp.zeros((2,), jnp.float32)])
    out = _run(tt, w2d, xcat32.reshape(-1), xcont_pad.reshape(-1), wc_pad)
    return out.reshape(bsz, 1)


# speedup vs baseline: 2.3924x; 2.3924x over previous
import jax
import jax.numpy as jnp
from jax import lax
from jax.experimental import pallas as pl
from jax.experimental.pallas import tpu as pltpu
from jax.experimental.pallas import tpu_sc as plsc

_F = 26
_V = 100000
_E = 32
_L = 16
_NC = 2
_NS = 16
_NW = _NC * _NS
_B = 16384
_RPW = _B // _NW
_FP = 32

_TCV = 2048
_TCC = 49
_VP = _TCC * _TCV
_SROWS = _F * _TCC * 16

_R = 64
_NCH = _RPW // _R
_CIDX = _R * _FP
_SG = 128
_NSG = _CIDX // _SG
_IDXW = _RPW * _FP


def _scores_body(tt_ref, w_ref, s_ref):
    t2 = tt_ref[0]
    wv = w_ref[0, :, :]
    sv = jnp.sum(t2 * wv, axis=0)
    s_ref[...] = sv.reshape(_TCV // 128, 128)


def _combine_body(s_ref, xcat_ref, xcont_ref, wc_ref, out_ref,
                  idx_v, g0, g1, xcont_v, out_v, wc_v, sem0, sem1):
    wid = lax.axis_index("s") * _NC + lax.axis_index("c")
    base = wid * _RPW

    pltpu.sync_copy(xcat_ref.at[pl.ds(base * _FP, _IDXW)], idx_v)
    pltpu.sync_copy(xcont_ref.at[pl.ds(base * _L, _RPW * _L)], xcont_v)
    pltpu.sync_copy(wc_ref, wc_v)

    lanes = lax.iota(jnp.int32, _L)

    def build_idx(j, carry):
        jg = j * _L + lanes
        f = jg & (_FP - 1)
        sl = pl.ds(j * _L, _L)
        t = idx_v[sl] + f * _VP
        idx_v[sl] = jnp.where(f < _F, t, wid * _IDXW + jg)
        return carry

    lax.fori_loop(0, _IDXW // _L, build_idx, 0)

    bufs = (g0, g1)
    sems = (sem0, sem1)

    def fire(ci, k):
        for g in range(_NSG):
            isl = pl.ds(ci * _CIDX + g * _SG, _SG)
            pltpu.async_copy(s_ref.at[idx_v.at[isl]],
                             bufs[k].at[pl.ds(g * _SG, _SG)], sems[k])

    def drain(ci, k):
        for g in range(_NSG):
            isl = pl.ds(ci * _CIDX + g * _SG, _SG)
            pltpu.make_async_copy(s_ref.at[idx_v.at[isl]],
                                  bufs[k].at[pl.ds(g * _SG, _SG)], sems[k]).wait()

    def lane_sum(v):
        for s in (8, 4, 2, 1):
            idx = (lanes ^ s)[:, None]
            dn = lax.GatherDimensionNumbers(
                offset_dims=(), collapsed_slice_dims=(0,), start_index_map=(0,))
            v = v + lax.gather(v, idx, dn, (1,),
                               mode=lax.GatherScatterMode.PROMISE_IN_BOUNDS)
        return v

    wc = wc_v[...]
    mask26 = lanes < (_F - _L)
    zero = jnp.zeros((_L,), jnp.float32)

    def compute(ci, k):
        rbuf = bufs[k]

        def group_body(g2, carry):
            def row_body(r2, tv):
                r = g2 * _L + r2
                v0 = rbuf[pl.ds(r * _FP, _L)]
                v1 = rbuf[pl.ds(r * _FP + _L, _L)]
                xc = xcont_v[pl.ds((ci * _R + r) * _L, _L)]
                acc = v0 + jnp.where(mask26, v1, zero) + xc * wc
                tot = lane_sum(acc)
                return jnp.where(lanes == r2, tot, tv)

            tv = lax.fori_loop(0, _L, row_body, zero)
            out_v[pl.ds(ci * _R + g2 * _L, _L)] = 1.0 / (1.0 + jnp.exp(-tv))
            return carry

        lax.fori_loop(0, _R // _L, group_body, 0)

    fire(0, 0)

    def outer(c2, carry):
        for k in range(2):
            i = c2 * 2 + k
            drain(i, k)

            @pl.when(i + 1 < _NCH)
            def _():
                fire(i + 1, k ^ 1)

            compute(i, k)
        return carry

    lax.fori_loop(0, _NCH // 2, outer, 0)

    pltpu.sync_copy(out_v, out_ref.at[pl.ds(base, _RPW)])


@jax.jit
def _run(tt, w2d, xcat32_flat, xcont_flat, wc_pad):
    s = pl.pallas_call(
        _scores_body,
        grid=(_F, _TCC),
        in_specs=[
            pl.BlockSpec((1, _E, _TCV), lambda f, c: (f, 0, c)),
            pl.BlockSpec((1, _E, 1), lambda f, c: (f, 0, 0)),
        ],
        out_specs=pl.BlockSpec((_TCV // 128, 128), lambda f, c: (f * _TCC + c, 0)),
        out_shape=jax.ShapeDtypeStruct((_SROWS, 128), jnp.float32),
    )(tt, w2d)

    combine = pl.kernel(
        _combine_body,
        out_type=jax.ShapeDtypeStruct((_B,), jnp.float32),
        mesh=plsc.VectorSubcoreMesh(core_axis_name="c", subcore_axis_name="s",
                                    num_cores=_NC, num_subcores=_NS),
        compiler_params=pltpu.CompilerParams(use_tc_tiling_on_sc=False),
        scratch_types=[
            pltpu.VMEM((_IDXW,), jnp.int32),
            pltpu.VMEM((_CIDX,), jnp.float32),
            pltpu.VMEM((_CIDX,), jnp.float32),
            pltpu.VMEM((_RPW * _L,), jnp.float32),
            pltpu.VMEM((_RPW,), jnp.float32),
            pltpu.VMEM((_L,), jnp.float32),
            pltpu.SemaphoreType.DMA,
            pltpu.SemaphoreType.DMA,
        ],
    )
    return combine(s.reshape(-1), xcat32_flat, xcont_flat, wc_pad)


def kernel(x_cat, x_cont, tables, W, b):
    bsz = x_cat.shape[0]
    tt = jnp.transpose(tables, (0, 2, 1))
    w2d = W[: _F * _E, 0].reshape(_F, _E, 1)
    xcat32 = jnp.concatenate(
        [x_cat, jnp.zeros((bsz, _FP - _F), jnp.int32)], axis=1)
    xcont_pad = jnp.concatenate(
        [x_cont, jnp.ones((bsz, 1), jnp.float32), jnp.zeros((bsz, 2), jnp.float32)],
        axis=1)
    wc_pad = jnp.concatenate([W[_F * _E:, 0], b, jnp.zeros((2,), jnp.float32)])
    out = _run(tt, w2d, xcat32.reshape(-1), xcont_pad.reshape(-1), wc_pad)
    return out.reshape(bsz, 1)


# speedup vs baseline: 5.6164x; 2.3476x over previous
import jax
import jax.numpy as jnp
from jax import lax
from jax.experimental import pallas as pl
from jax.experimental.pallas import tpu as pltpu
from jax.experimental.pallas import tpu_sc as plsc

_F = 26
_V = 100000
_E = 32
_L = 16
_NC = 2
_NS = 16
_NW = _NC * _NS
_B = 16384
_RPW = _B // _NW
_FP = 32

_TCV = 8192
_TCC = 13
_VP = _TCC * _TCV
_SROWS = _F * _TCC * (_TCV // 128)

_R = 64
_NCH = _RPW // _R
_CIDX = _R * _FP
_SG = 128
_NSG = _CIDX // _SG
_IDXW = _RPW * _FP


def _scores_body(tt_ref, w_ref, s_ref):
    t2 = tt_ref[0]
    wv = w_ref[0, :, :]
    sv = jnp.sum(t2 * wv, axis=0)
    s_ref[...] = sv.reshape(_TCV // 128, 128)


def _combine_body(s_ref, xcat_ref, xcont_ref, wc_ref, out_ref,
                  idx_v, g0, g1, xcont_v, out_v, wc_v, sem0, sem1):
    wid = lax.axis_index("s") * _NC + lax.axis_index("c")
    base = wid * _RPW

    pltpu.sync_copy(xcat_ref.at[pl.ds(base * _FP, _IDXW)], idx_v)
    pltpu.sync_copy(xcont_ref.at[pl.ds(base * _L, _RPW * _L)], xcont_v)
    pltpu.sync_copy(wc_ref, wc_v)

    lanes = lax.iota(jnp.int32, _L)

    def build_idx(j, carry):
        jg = j * _L + lanes
        f = jg & (_FP - 1)
        sl = pl.ds(j * _L, _L)
        t = idx_v[sl] + f * _VP
        idx_v[sl] = jnp.where(f < _F, t, wid * _IDXW + jg)
        return carry

    lax.fori_loop(0, _IDXW // _L, build_idx, 0)

    bufs = (g0, g1)
    sems = (sem0, sem1)

    def fire(ci, k):
        for g in range(_NSG):
            isl = pl.ds(ci * _CIDX + g * _SG, _SG)
            pltpu.async_copy(s_ref.at[idx_v.at[isl]],
                             bufs[k].at[pl.ds(g * _SG, _SG)], sems[k])

    def drain(ci, k):
        for g in range(_NSG):
            isl = pl.ds(ci * _CIDX + g * _SG, _SG)
            pltpu.make_async_copy(s_ref.at[idx_v.at[isl]],
                                  bufs[k].at[pl.ds(g * _SG, _SG)], sems[k]).wait()

    def lane_sum(v):
        for s in (8, 4, 2, 1):
            idx = (lanes ^ s)[:, None]
            dn = lax.GatherDimensionNumbers(
                offset_dims=(), collapsed_slice_dims=(0,), start_index_map=(0,))
            v = v + lax.gather(v, idx, dn, (1,),
                               mode=lax.GatherScatterMode.PROMISE_IN_BOUNDS)
        return v

    wc = wc_v[...]
    mask26 = lanes < (_F - _L)
    zero = jnp.zeros((_L,), jnp.float32)

    def compute(ci, k):
        rbuf = bufs[k]

        def group_body(g2, carry):
            def row_body(r2, tv):
                r = g2 * _L + r2
                v0 = rbuf[pl.ds(r * _FP, _L)]
                v1 = rbuf[pl.ds(r * _FP + _L, _L)]
                xc = xcont_v[pl.ds((ci * _R + r) * _L, _L)]
                acc = v0 + jnp.where(mask26, v1, zero) + xc * wc
                tot = lane_sum(acc)
                return jnp.where(lanes == r2, tot, tv)

            tv = lax.fori_loop(0, _L, row_body, zero)
            out_v[pl.ds(ci * _R + g2 * _L, _L)] = 1.0 / (1.0 + jnp.exp(-tv))
            return carry

        lax.fori_loop(0, _R // _L, group_body, 0)

    fire(0, 0)

    def outer(c2, carry):
        for k in range(2):
            i = c2 * 2 + k
            drain(i, k)

            @pl.when(i + 1 < _NCH)
            def _():
                fire(i + 1, k ^ 1)

            compute(i, k)
        return carry

    lax.fori_loop(0, _NCH // 2, outer, 0)

    pltpu.sync_copy(out_v, out_ref.at[pl.ds(base, _RPW)])


@jax.jit
def _run(tt, w2d, xcat32_flat, xcont_flat, wc_pad):
    s = pl.pallas_call(
        _scores_body,
        grid=(_F, _TCC),
        in_specs=[
            pl.BlockSpec((1, _E, _TCV), lambda f, c: (f, 0, c)),
            pl.BlockSpec((1, _E, 1), lambda f, c: (f, 0, 0)),
        ],
        out_specs=pl.BlockSpec((_TCV // 128, 128), lambda f, c: (f * _TCC + c, 0)),
        out_shape=jax.ShapeDtypeStruct((_SROWS, 128), jnp.float32),
    )(tt, w2d)

    combine = pl.kernel(
        _combine_body,
        out_type=jax.ShapeDtypeStruct((_B,), jnp.float32),
        mesh=plsc.VectorSubcoreMesh(core_axis_name="c", subcore_axis_name="s",
                                    num_cores=_NC, num_subcores=_NS),
        compiler_params=pltpu.CompilerParams(use_tc_tiling_on_sc=False),
        scratch_types=[
            pltpu.VMEM((_IDXW,), jnp.int32),
            pltpu.VMEM((_CIDX,), jnp.float32),
            pltpu.VMEM((_CIDX,), jnp.float32),
            pltpu.VMEM((_RPW * _L,), jnp.float32),
            pltpu.VMEM((_RPW,), jnp.float32),
            pltpu.VMEM((_L,), jnp.float32),
            pltpu.SemaphoreType.DMA,
            pltpu.SemaphoreType.DMA,
        ],
    )
    return combine(s.reshape(-1), xcat32_flat, xcont_flat, wc_pad)


def kernel(x_cat, x_cont, tables, W, b):
    bsz = x_cat.shape[0]
    tt = jnp.transpose(tables, (0, 2, 1))
    w2d = W[: _F * _E, 0].reshape(_F, _E, 1)
    xcat32 = jnp.concatenate(
        [x_cat, jnp.zeros((bsz, _FP - _F), jnp.int32)], axis=1)
    xcont_pad = jnp.concatenate(
        [x_cont, jnp.ones((bsz, 1), jnp.float32), jnp.zeros((bsz, 2), jnp.float32)],
        axis=1)
    wc_pad = jnp.concatenate([W[_F * _E:, 0], b, jnp.zeros((2,), jnp.float32)])
    out = _run(tt, w2d, xcat32.reshape(-1), xcont_pad.reshape(-1), wc_pad)
    return out.reshape(bsz, 1)


# speedup vs baseline: 7.2021x; 1.2823x over previous
import jax
import jax.numpy as jnp
from jax import lax
from jax.experimental import pallas as pl
from jax.experimental.pallas import tpu as pltpu
from jax.experimental.pallas import tpu_sc as plsc

_F = 26
_V = 100000
_E = 32
_L = 16
_NC = 2
_NS = 16
_NW = _NC * _NS
_B = 16384
_RPW = _B // _NW
_FP = 32

_TCV = 16384
_TCC = 7
_VP = _TCC * _TCV
_SROWS = _F * _TCC * (_TCV // 128)

_R = 64
_NCH = _RPW // _R
_CIDX = _R * _FP
_SG = 128
_NSG = _CIDX // _SG
_IDXW = _RPW * _FP


def _scores_body(tt_ref, w_ref, s_ref):
    t2 = tt_ref[0]
    wv = w_ref[0, :, :]
    sv = jnp.sum(t2 * wv, axis=0)
    s_ref[...] = sv.reshape(_TCV // 128, 128)


def _combine_body(s_ref, xcat_ref, xcont_ref, wc_ref, out_ref,
                  idx_v, g0, g1, xcont_v, out_v, wc_v, sem0, sem1):
    wid = lax.axis_index("s") * _NC + lax.axis_index("c")
    base = wid * _RPW

    pltpu.sync_copy(xcat_ref.at[pl.ds(base * _FP, _IDXW)], idx_v)
    pltpu.sync_copy(xcont_ref.at[pl.ds(base * _L, _RPW * _L)], xcont_v)
    pltpu.sync_copy(wc_ref, wc_v)

    lanes = lax.iota(jnp.int32, _L)

    def build_idx(j, carry):
        jg = j * _L + lanes
        f = jg & (_FP - 1)
        sl = pl.ds(j * _L, _L)
        t = idx_v[sl] + f * _VP
        idx_v[sl] = jnp.where(f < _F, t, wid * _IDXW + jg)
        return carry

    lax.fori_loop(0, _IDXW // _L, build_idx, 0)

    bufs = (g0, g1)
    sems = (sem0, sem1)

    def fire(ci, k):
        for g in range(_NSG):
            isl = pl.ds(ci * _CIDX + g * _SG, _SG)
            pltpu.async_copy(s_ref.at[idx_v.at[isl]],
                             bufs[k].at[pl.ds(g * _SG, _SG)], sems[k])

    def drain(ci, k):
        for g in range(_NSG):
            isl = pl.ds(ci * _CIDX + g * _SG, _SG)
            pltpu.make_async_copy(s_ref.at[idx_v.at[isl]],
                                  bufs[k].at[pl.ds(g * _SG, _SG)], sems[k]).wait()

    def lane_sum(v):
        for s in (8, 4, 2, 1):
            idx = (lanes ^ s)[:, None]
            dn = lax.GatherDimensionNumbers(
                offset_dims=(), collapsed_slice_dims=(0,), start_index_map=(0,))
            v = v + lax.gather(v, idx, dn, (1,),
                               mode=lax.GatherScatterMode.PROMISE_IN_BOUNDS)
        return v

    wc = wc_v[...]
    mask26 = lanes < (_F - _L)
    zero = jnp.zeros((_L,), jnp.float32)

    def compute(ci, k):
        rbuf = bufs[k]

        def group_body(g2, carry):
            def row_body(r2, tv):
                r = g2 * _L + r2
                v0 = rbuf[pl.ds(r * _FP, _L)]
                v1 = rbuf[pl.ds(r * _FP + _L, _L)]
                xc = xcont_v[pl.ds((ci * _R + r) * _L, _L)]
                acc = v0 + jnp.where(mask26, v1, zero) + xc * wc
                tot = lane_sum(acc)
                return jnp.where(lanes == r2, tot, tv)

            tv = lax.fori_loop(0, _L, row_body, zero)
            out_v[pl.ds(ci * _R + g2 * _L, _L)] = 1.0 / (1.0 + jnp.exp(-tv))
            return carry

        lax.fori_loop(0, _R // _L, group_body, 0)

    fire(0, 0)

    def outer(c2, carry):
        for k in range(2):
            i = c2 * 2 + k
            drain(i, k)

            @pl.when(i + 1 < _NCH)
            def _():
                fire(i + 1, k ^ 1)

            compute(i, k)
        return carry

    lax.fori_loop(0, _NCH // 2, outer, 0)

    pltpu.sync_copy(out_v, out_ref.at[pl.ds(base, _RPW)])


@jax.jit
def _run(tt, w2d, xcat32_flat, xcont_flat, wc_pad):
    s = pl.pallas_call(
        _scores_body,
        grid=(_F, _TCC),
        in_specs=[
            pl.BlockSpec((1, _E, _TCV), lambda f, c: (f, 0, c)),
            pl.BlockSpec((1, _E, 1), lambda f, c: (f, 0, 0)),
        ],
        out_specs=pl.BlockSpec((_TCV // 128, 128), lambda f, c: (f * _TCC + c, 0)),
        out_shape=jax.ShapeDtypeStruct((_SROWS, 128), jnp.float32),
    )(tt, w2d)

    combine = pl.kernel(
        _combine_body,
        out_type=jax.ShapeDtypeStruct((_B,), jnp.float32),
        mesh=plsc.VectorSubcoreMesh(core_axis_name="c", subcore_axis_name="s",
                                    num_cores=_NC, num_subcores=_NS),
        compiler_params=pltpu.CompilerParams(use_tc_tiling_on_sc=False),
        scratch_types=[
            pltpu.VMEM((_IDXW,), jnp.int32),
            pltpu.VMEM((_CIDX,), jnp.float32),
            pltpu.VMEM((_CIDX,), jnp.float32),
            pltpu.VMEM((_RPW * _L,), jnp.float32),
            pltpu.VMEM((_RPW,), jnp.float32),
            pltpu.VMEM((_L,), jnp.float32),
            pltpu.SemaphoreType.DMA,
            pltpu.SemaphoreType.DMA,
        ],
    )
    return combine(s.reshape(-1), xcat32_flat, xcont_flat, wc_pad)


def kernel(x_cat, x_cont, tables, W, b):
    bsz = x_cat.shape[0]
    tt = jnp.transpose(tables, (0, 2, 1))
    w2d = W[: _F * _E, 0].reshape(_F, _E, 1)
    xcat32 = jnp.concatenate(
        [x_cat, jnp.zeros((bsz, _FP - _F), jnp.int32)], axis=1)
    xcont_pad = jnp.concatenate(
        [x_cont, jnp.ones((bsz, 1), jnp.float32), jnp.zeros((bsz, 2), jnp.float32)],
        axis=1)
    wc_pad = jnp.concatenate([W[_F * _E:, 0], b, jnp.zeros((2,), jnp.float32)])
    out = _run(tt, w2d, xcat32.reshape(-1), xcont_pad.reshape(-1), wc_pad)
    return out.reshape(bsz, 1)


# speedup vs baseline: 7.8849x; 1.0948x over previous
import jax
import jax.numpy as jnp
from jax import lax
from jax.experimental import pallas as pl
from jax.experimental.pallas import tpu as pltpu
from jax.experimental.pallas import tpu_sc as plsc

_F = 26
_V = 100000
_E = 32
_L = 16
_NC = 2
_NS = 16
_NW = _NC * _NS
_B = 16384
_RPW = _B // _NW
_FP = 32

_TCV = 32768
_TCC = 4
_VP = _TCC * _TCV
_SROWS = _F * _TCC * (_TCV // 128)

_R = 64
_NCH = _RPW // _R
_CIDX = _R * _FP
_SG = 128
_NSG = _CIDX // _SG
_IDXW = _RPW * _FP


def _scores_body(tt_ref, w_ref, s_ref):
    t2 = tt_ref[0]
    wv = w_ref[0, :, :]
    sv = jnp.sum(t2 * wv, axis=0)
    s_ref[...] = sv.reshape(_TCV // 128, 128)


def _combine_body(s_ref, xcat_ref, xcont_ref, wc_ref, out_ref,
                  idx_v, g0, g1, xcont_v, out_v, wc_v, sem0, sem1):
    wid = lax.axis_index("s") * _NC + lax.axis_index("c")
    base = wid * _RPW

    pltpu.sync_copy(xcat_ref.at[pl.ds(base * _FP, _IDXW)], idx_v)
    pltpu.sync_copy(xcont_ref.at[pl.ds(base * _L, _RPW * _L)], xcont_v)
    pltpu.sync_copy(wc_ref, wc_v)

    lanes = lax.iota(jnp.int32, _L)

    def build_idx(j, carry):
        jg = j * _L + lanes
        f = jg & (_FP - 1)
        sl = pl.ds(j * _L, _L)
        t = idx_v[sl] + f * _VP
        idx_v[sl] = jnp.where(f < _F, t, wid * _IDXW + jg)
        return carry

    lax.fori_loop(0, _IDXW // _L, build_idx, 0)

    bufs = (g0, g1)
    sems = (sem0, sem1)

    def fire(ci, k):
        for g in range(_NSG):
            isl = pl.ds(ci * _CIDX + g * _SG, _SG)
            pltpu.async_copy(s_ref.at[idx_v.at[isl]],
                             bufs[k].at[pl.ds(g * _SG, _SG)], sems[k])

    def drain(ci, k):
        for g in range(_NSG):
            isl = pl.ds(ci * _CIDX + g * _SG, _SG)
            pltpu.make_async_copy(s_ref.at[idx_v.at[isl]],
                                  bufs[k].at[pl.ds(g * _SG, _SG)], sems[k]).wait()

    def lane_sum(v):
        for s in (8, 4, 2, 1):
            idx = (lanes ^ s)[:, None]
            dn = lax.GatherDimensionNumbers(
                offset_dims=(), collapsed_slice_dims=(0,), start_index_map=(0,))
            v = v + lax.gather(v, idx, dn, (1,),
                               mode=lax.GatherScatterMode.PROMISE_IN_BOUNDS)
        return v

    wc = wc_v[...]
    mask26 = lanes < (_F - _L)
    zero = jnp.zeros((_L,), jnp.float32)

    def compute(ci, k):
        rbuf = bufs[k]

        def group_body(g2, carry):
            def row_body(r2, tv):
                r = g2 * _L + r2
                v0 = rbuf[pl.ds(r * _FP, _L)]
                v1 = rbuf[pl.ds(r * _FP + _L, _L)]
                xc = xcont_v[pl.ds((ci * _R + r) * _L, _L)]
                acc = v0 + jnp.where(mask26, v1, zero) + xc * wc
                tot = lane_sum(acc)
                return jnp.where(lanes == r2, tot, tv)

            tv = lax.fori_loop(0, _L, row_body, zero)
            out_v[pl.ds(ci * _R + g2 * _L, _L)] = 1.0 / (1.0 + jnp.exp(-tv))
            return carry

        lax.fori_loop(0, _R // _L, group_body, 0)

    fire(0, 0)

    def outer(c2, carry):
        for k in range(2):
            i = c2 * 2 + k
            drain(i, k)

            @pl.when(i + 1 < _NCH)
            def _():
                fire(i + 1, k ^ 1)

            compute(i, k)
        return carry

    lax.fori_loop(0, _NCH // 2, outer, 0)

    pltpu.sync_copy(out_v, out_ref.at[pl.ds(base, _RPW)])


@jax.jit
def _run(tt, w2d, xcat32_flat, xcont_flat, wc_pad):
    s = pl.pallas_call(
        _scores_body,
        grid=(_F, _TCC),
        in_specs=[
            pl.BlockSpec((1, _E, _TCV), lambda f, c: (f, 0, c)),
            pl.BlockSpec((1, _E, 1), lambda f, c: (f, 0, 0)),
        ],
        out_specs=pl.BlockSpec((_TCV // 128, 128), lambda f, c: (f * _TCC + c, 0)),
        out_shape=jax.ShapeDtypeStruct((_SROWS, 128), jnp.float32),
    )(tt, w2d)

    combine = pl.kernel(
        _combine_body,
        out_type=jax.ShapeDtypeStruct((_B,), jnp.float32),
        mesh=plsc.VectorSubcoreMesh(core_axis_name="c", subcore_axis_name="s",
                                    num_cores=_NC, num_subcores=_NS),
        compiler_params=pltpu.CompilerParams(use_tc_tiling_on_sc=False),
        scratch_types=[
            pltpu.VMEM((_IDXW,), jnp.int32),
            pltpu.VMEM((_CIDX,), jnp.float32),
            pltpu.VMEM((_CIDX,), jnp.float32),
            pltpu.VMEM((_RPW * _L,), jnp.float32),
            pltpu.VMEM((_RPW,), jnp.float32),
            pltpu.VMEM((_L,), jnp.float32),
            pltpu.SemaphoreType.DMA,
            pltpu.SemaphoreType.DMA,
        ],
    )
    return combine(s.reshape(-1), xcat32_flat, xcont_flat, wc_pad)


def kernel(x_cat, x_cont, tables, W, b):
    bsz = x_cat.shape[0]
    tt = jnp.transpose(tables, (0, 2, 1))
    w2d = W[: _F * _E, 0].reshape(_F, _E, 1)
    xcat32 = jnp.concatenate(
        [x_cat, jnp.zeros((bsz, _FP - _F), jnp.int32)], axis=1)
    xcont_pad = jnp.concatenate(
        [x_cont, jnp.ones((bsz, 1), jnp.float32), jnp.zeros((bsz, 2), jnp.float32)],
        axis=1)
    wc_pad = jnp.concatenate([W[_F * _E:, 0], b, jnp.zeros((2,), jnp.float32)])
    out = _run(tt, w2d, xcat32.reshape(-1), xcont_pad.reshape(-1), wc_pad)
    return out.reshape(bsz, 1)


# speedup vs baseline: 10.3237x; 1.3093x over previous
import jax
import jax.numpy as jnp
from jax import lax
from jax.experimental import pallas as pl
from jax.experimental.pallas import tpu as pltpu
from jax.experimental.pallas import tpu_sc as plsc

_F = 26
_V = 100000
_E = 32
_L = 16
_NC = 2
_NS = 16
_NW = _NC * _NS
_B = 16384
_RPW = _B // _NW
_FP = 32

_TCV = 100352
_TCC = 1
_VP = _TCC * _TCV
_SROWS = _F * _TCC * (_TCV // 128)

_R = 64
_NCH = _RPW // _R
_CIDX = _R * _FP
_SG = 128
_NSG = _CIDX // _SG
_IDXW = _RPW * _FP


def _scores_body(tt_ref, w_ref, s_ref):
    t2 = tt_ref[0]
    wv = w_ref[0, :, :]
    sv = jnp.sum(t2 * wv, axis=0)
    s_ref[...] = sv.reshape(_TCV // 128, 128)


def _combine_body(s_ref, xcat_ref, xcont_ref, wc_ref, out_ref,
                  idx_v, g0, g1, xcont_v, out_v, wc_v, sem0, sem1):
    wid = lax.axis_index("s") * _NC + lax.axis_index("c")
    base = wid * _RPW

    pltpu.sync_copy(xcat_ref.at[pl.ds(base * _FP, _IDXW)], idx_v)
    pltpu.sync_copy(xcont_ref.at[pl.ds(base * _L, _RPW * _L)], xcont_v)
    pltpu.sync_copy(wc_ref, wc_v)

    lanes = lax.iota(jnp.int32, _L)

    def build_idx(j, carry):
        jg = j * _L + lanes
        f = jg & (_FP - 1)
        sl = pl.ds(j * _L, _L)
        t = idx_v[sl] + f * _VP
        idx_v[sl] = jnp.where(f < _F, t, wid * _IDXW + jg)
        return carry

    lax.fori_loop(0, _IDXW // _L, build_idx, 0)

    bufs = (g0, g1)
    sems = (sem0, sem1)

    def fire(ci, k):
        for g in range(_NSG):
            isl = pl.ds(ci * _CIDX + g * _SG, _SG)
            pltpu.async_copy(s_ref.at[idx_v.at[isl]],
                             bufs[k].at[pl.ds(g * _SG, _SG)], sems[k])

    def drain(ci, k):
        for g in range(_NSG):
            isl = pl.ds(ci * _CIDX + g * _SG, _SG)
            pltpu.make_async_copy(s_ref.at[idx_v.at[isl]],
                                  bufs[k].at[pl.ds(g * _SG, _SG)], sems[k]).wait()

    def lane_sum(v):
        for s in (8, 4, 2, 1):
            idx = (lanes ^ s)[:, None]
            dn = lax.GatherDimensionNumbers(
                offset_dims=(), collapsed_slice_dims=(0,), start_index_map=(0,))
            v = v + lax.gather(v, idx, dn, (1,),
                               mode=lax.GatherScatterMode.PROMISE_IN_BOUNDS)
        return v

    wc = wc_v[...]
    mask26 = lanes < (_F - _L)
    zero = jnp.zeros((_L,), jnp.float32)

    def compute(ci, k):
        rbuf = bufs[k]

        def group_body(g2, carry):
            def row_body(r2, tv):
                r = g2 * _L + r2
                v0 = rbuf[pl.ds(r * _FP, _L)]
                v1 = rbuf[pl.ds(r * _FP + _L, _L)]
                xc = xcont_v[pl.ds((ci * _R + r) * _L, _L)]
                acc = v0 + jnp.where(mask26, v1, zero) + xc * wc
                tot = lane_sum(acc)
                return jnp.where(lanes == r2, tot, tv)

            tv = lax.fori_loop(0, _L, row_body, zero)
            out_v[pl.ds(ci * _R + g2 * _L, _L)] = 1.0 / (1.0 + jnp.exp(-tv))
            return carry

        lax.fori_loop(0, _R // _L, group_body, 0)

    fire(0, 0)

    def outer(c2, carry):
        for k in range(2):
            i = c2 * 2 + k
            drain(i, k)

            @pl.when(i + 1 < _NCH)
            def _():
                fire(i + 1, k ^ 1)

            compute(i, k)
        return carry

    lax.fori_loop(0, _NCH // 2, outer, 0)

    pltpu.sync_copy(out_v, out_ref.at[pl.ds(base, _RPW)])


@jax.jit
def _run(tt, w2d, xcat32_flat, xcont_flat, wc_pad):
    s = pl.pallas_call(
        _scores_body,
        grid=(_F, _TCC),
        in_specs=[
            pl.BlockSpec((1, _E, _TCV), lambda f, c: (f, 0, c)),
            pl.BlockSpec((1, _E, 1), lambda f, c: (f, 0, 0)),
        ],
        out_specs=pl.BlockSpec((_TCV // 128, 128), lambda f, c: (f * _TCC + c, 0)),
        out_shape=jax.ShapeDtypeStruct((_SROWS, 128), jnp.float32),
    )(tt, w2d)

    combine = pl.kernel(
        _combine_body,
        out_type=jax.ShapeDtypeStruct((_B,), jnp.float32),
        mesh=plsc.VectorSubcoreMesh(core_axis_name="c", subcore_axis_name="s",
                                    num_cores=_NC, num_subcores=_NS),
        compiler_params=pltpu.CompilerParams(use_tc_tiling_on_sc=False),
        scratch_types=[
            pltpu.VMEM((_IDXW,), jnp.int32),
            pltpu.VMEM((_CIDX,), jnp.float32),
            pltpu.VMEM((_CIDX,), jnp.float32),
            pltpu.VMEM((_RPW * _L,), jnp.float32),
            pltpu.VMEM((_RPW,), jnp.float32),
            pltpu.VMEM((_L,), jnp.float32),
            pltpu.SemaphoreType.DMA,
            pltpu.SemaphoreType.DMA,
        ],
    )
    return combine(s.reshape(-1), xcat32_flat, xcont_flat, wc_pad)


def kernel(x_cat, x_cont, tables, W, b):
    bsz = x_cat.shape[0]
    tt = jnp.transpose(tables, (0, 2, 1))
    w2d = W[: _F * _E, 0].reshape(_F, _E, 1)
    xcat32 = jnp.concatenate(
        [x_cat, jnp.zeros((bsz, _FP - _F), jnp.int32)], axis=1)
    xcont_pad = jnp.concatenate(
        [x_cont, jnp.ones((bsz, 1), jnp.float32), jnp.zeros((bsz, 2), jnp.float32)],
        axis=1)
    wc_pad = jnp.concatenate([W[_F * _E:, 0], b, jnp.zeros((2,), jnp.float32)])
    out = _run(tt, w2d, xcat32.reshape(-1), xcont_pad.reshape(-1), wc_pad)
    return out.reshape(bsz, 1)
